# baseline (speedup 1.0000x reference)
import jax
import jax.numpy as jnp
from jax.experimental import pallas as pl
from jax.experimental.pallas import tpu as pltpu
from jax.experimental.pallas import tpu_sc as plsc

_DIM = 64
_NEMB = 1024
_ROWS_PER_BLOCK = 1024
_GATHER_WINDOW = 256


def _distance_argmin_body(x_ref, e_ref, idx_ref, acc_ref):
    i = pl.program_id(0)
    xb = x_ref[...]
    emb = e_ref[...]
    neg2m = jax.lax.dot_general(
        xb * -2.0, emb,
        dimension_numbers=(((1,), (0,)), ((), ())),
        preferred_element_type=jnp.float32,
    )
    x2 = jnp.sum(xb * xb, axis=1, keepdims=True)
    e2 = jnp.sum(emb * emb, axis=0, keepdims=True)
    d = (x2 + e2) + neg2m
    m = jnp.min(d, axis=1, keepdims=True)
    lane = jax.lax.broadcasted_iota(jnp.int32, d.shape, 1)
    idx = jnp.min(jnp.where(d == m, lane, jnp.int32(1 << 30)), axis=1)
    idx_ref[...] = idx.astype(jnp.int32)

    @pl.when(i == 0)
    def _():
        acc_ref[...] = jnp.zeros_like(acc_ref)

    acc_ref[...] += jnp.full(acc_ref.shape, jnp.sum(m), dtype=jnp.float32)


def _distance_argmin(flat_x, embeddings):
    n = flat_x.shape[0]
    nblk = n // _ROWS_PER_BLOCK
    return pl.pallas_call(
        _distance_argmin_body,
        grid=(nblk,),
        in_specs=[
            pl.BlockSpec((_ROWS_PER_BLOCK, _DIM), lambda i: (i, 0)),
            pl.BlockSpec((_DIM, _NEMB), lambda i: (0, 0)),
        ],
        out_specs=[
            pl.BlockSpec((_ROWS_PER_BLOCK,), lambda i: (i,)),
            pl.BlockSpec((8, 128), lambda i: (0, 0)),
        ],
        out_shape=[
            jax.ShapeDtypeStruct((n,), jnp.int32),
            jax.ShapeDtypeStruct((8, 128), jnp.float32),
        ],
        compiler_params=pltpu.CompilerParams(
            dimension_semantics=("arbitrary",)),
    )(flat_x, embeddings)


def _sc_gather(table, indices):
    n = indices.shape[0]
    width = table.shape[1]
    mesh = plsc.VectorSubcoreMesh(
        core_axis_name="core", subcore_axis_name="subcore")
    num_workers = mesh.num_cores * mesh.num_subcores
    per_worker = n // num_workers
    chunk = min(per_worker, 512)

    @pl.kernel(
        out_type=jax.ShapeDtypeStruct((n, width), jnp.float32),
        mesh=mesh,
        scratch_types=[
            pltpu.VMEM((chunk,), jnp.int32),
            pltpu.VMEM((chunk, width), jnp.float32),
            pltpu.SemaphoreType.DMA,
        ],
    )
    def gather_kernel(tab_hbm, i_hbm, o_hbm, idx_v, rows_v, sem):
        wid = (jax.lax.axis_index("subcore") * mesh.num_cores
               + jax.lax.axis_index("core"))
        for c in range(per_worker // chunk):
            base = wid * per_worker + c * chunk
            pltpu.sync_copy(i_hbm.at[pl.ds(base, chunk)], idx_v)
            pltpu.async_copy(tab_hbm.at[idx_v], rows_v, sem).wait()
            pltpu.sync_copy(rows_v, o_hbm.at[pl.ds(base, chunk)])

    return gather_kernel(table, indices)


def kernel(x, embeddings):
    n = x.shape[0] * x.shape[1]
    flat_x = x.reshape(n, _DIM)
    idx, acc = _distance_argmin(flat_x, embeddings)
    loss = acc[0, 0] * (1.25 / (n * _DIM))
    table = jnp.pad(embeddings.T, ((0, 0), (0, 128 - _DIM)))
    quantized = _sc_gather(table, idx)[:, :_DIM]
    return quantized.reshape(x.shape), loss


# speedup vs baseline: 1.1302x; 1.1302x over previous
import jax
import jax.numpy as jnp
from jax.experimental import pallas as pl
from jax.experimental.pallas import tpu as pltpu
from jax.experimental.pallas import tpu_sc as plsc

_DIM = 64
_NEMB = 1024
_ROWS_PER_BLOCK = 1024
_GATHER_WINDOW = 256


def _distance_argmin_body(x_ref, e_ref, idx_ref, acc_ref):
    i = pl.program_id(0)
    xb = x_ref[...]
    emb = e_ref[...]
    neg2m = jax.lax.dot_general(
        xb * -2.0, emb,
        dimension_numbers=(((1,), (0,)), ((), ())),
        preferred_element_type=jnp.float32,
    )
    x2 = jnp.sum(xb * xb, axis=1, keepdims=True)
    e2 = jnp.sum(emb * emb, axis=0, keepdims=True)
    d = (x2 + e2) + neg2m
    m = jnp.min(d, axis=1, keepdims=True)
    lane = jax.lax.broadcasted_iota(jnp.int32, d.shape, 1)
    idx = jnp.min(jnp.where(d == m, lane, jnp.int32(1 << 30)), axis=1)
    idx_ref[...] = idx.astype(jnp.int32)

    @pl.when(i == 0)
    def _():
        acc_ref[...] = jnp.zeros_like(acc_ref)

    acc_ref[...] += jnp.full(acc_ref.shape, jnp.sum(m), dtype=jnp.float32)


def _distance_argmin(flat_x, embeddings):
    n = flat_x.shape[0]
    nblk = n // _ROWS_PER_BLOCK
    return pl.pallas_call(
        _distance_argmin_body,
        grid=(nblk,),
        in_specs=[
            pl.BlockSpec((_ROWS_PER_BLOCK, _DIM), lambda i: (i, 0)),
            pl.BlockSpec((_DIM, _NEMB), lambda i: (0, 0)),
        ],
        out_specs=[
            pl.BlockSpec((_ROWS_PER_BLOCK,), lambda i: (i,)),
            pl.BlockSpec((8, 128), lambda i: (0, 0)),
        ],
        out_shape=[
            jax.ShapeDtypeStruct((n,), jnp.int32),
            jax.ShapeDtypeStruct((8, 128), jnp.float32),
        ],
        compiler_params=pltpu.CompilerParams(
            dimension_semantics=("arbitrary",)),
    )(flat_x, embeddings)


def _sc_gather(table, indices):
    n = indices.shape[0]
    width = table.shape[1]
    mesh = plsc.VectorSubcoreMesh(
        core_axis_name="core", subcore_axis_name="subcore")
    num_workers = mesh.num_cores * mesh.num_subcores
    per_worker = n // num_workers

    @pl.kernel(
        out_type=jax.ShapeDtypeStruct((n, width), jnp.float32),
        mesh=mesh,
        scratch_types=[
            pltpu.VMEM((per_worker,), jnp.int32),
            pltpu.VMEM((per_worker, width), jnp.float32),
            pltpu.SemaphoreType.DMA,
        ],
        compiler_params=pltpu.CompilerParams(use_tc_tiling_on_sc=False),
    )
    def gather_kernel(tab_hbm, i_hbm, o_hbm, idx_v, rows_v, sem):
        wid = (jax.lax.axis_index("subcore") * mesh.num_cores
               + jax.lax.axis_index("core"))
        base = wid * per_worker
        pltpu.sync_copy(i_hbm.at[pl.ds(base, per_worker)], idx_v)
        pltpu.async_copy(tab_hbm.at[idx_v], rows_v, sem).wait()
        pltpu.sync_copy(rows_v, o_hbm.at[pl.ds(base, per_worker)])

    return gather_kernel(table, indices)


def kernel(x, embeddings):
    n = x.shape[0] * x.shape[1]
    flat_x = x.reshape(n, _DIM)
    table = embeddings.T
    nchunks = 2
    half = n // nchunks
    parts, accs = [], []
    for c in range(nchunks):
        idx_c, acc_c = _distance_argmin(
            jax.lax.slice_in_dim(flat_x, c * half, (c + 1) * half, axis=0),
            embeddings)
        parts.append(_sc_gather(table, idx_c))
        accs.append(acc_c[0, 0])
    loss = sum(accs) * (1.25 / (n * _DIM))
    quantized = jnp.concatenate(parts, axis=0)
    return quantized.reshape(x.shape), loss


# speedup vs baseline: 1.2826x; 1.1348x over previous
import jax
import jax.numpy as jnp
from jax.experimental import pallas as pl
from jax.experimental.pallas import tpu as pltpu
from jax.experimental.pallas import tpu_sc as plsc

_DIM = 64
_NEMB = 1024
_ROWS_PER_BLOCK = 512
_GATHER_WINDOW = 256


_BIG = 3.0e38


def _distance_argmin_body(x_ref, e_ref, idx_ref, acc_ref):
    i = pl.program_id(0)
    xb = x_ref[...]
    emb = e_ref[...]
    neg2m = jax.lax.dot_general(
        xb * -2.0, emb,
        dimension_numbers=(((1,), (0,)), ((), ())),
        preferred_element_type=jnp.float32,
    )
    x2 = jnp.sum(xb * xb, axis=1, keepdims=True)
    e2 = jnp.sum(emb * emb, axis=0, keepdims=True)
    x2b = jnp.broadcast_to(x2, (_ROWS_PER_BLOCK, 128))
    lane = jax.lax.broadcasted_iota(
        jnp.int32, (_ROWS_PER_BLOCK, 128), 1).astype(jnp.float32)
    run_min = jnp.full((_ROWS_PER_BLOCK, 128), _BIG, jnp.float32)
    run_idx = jnp.full((_ROWS_PER_BLOCK, 128), _BIG, jnp.float32)
    for g in range(_NEMB // 128):
        lo, hi = g * 128, (g + 1) * 128
        dg = (x2b + e2[:, lo:hi]) + neg2m[:, lo:hi]
        better = dg < run_min
        run_idx = jnp.where(better, lane + jnp.float32(g * 128), run_idx)
        run_min = jnp.minimum(dg, run_min)
    m = jnp.min(run_min, axis=1, keepdims=True)
    idxf = jnp.min(jnp.where(run_min == m, run_idx, _BIG), axis=1)
    idx_ref[...] = idxf.astype(jnp.int32)

    @pl.when(i == 0)
    def _():
        acc_ref[...] = jnp.zeros_like(acc_ref)

    acc_ref[...] += jnp.full(acc_ref.shape, jnp.sum(m), dtype=jnp.float32)


def _distance_argmin(flat_x, embeddings):
    n = flat_x.shape[0]
    nblk = n // _ROWS_PER_BLOCK
    return pl.pallas_call(
        _distance_argmin_body,
        grid=(nblk,),
        in_specs=[
            pl.BlockSpec((_ROWS_PER_BLOCK, _DIM), lambda i: (i, 0)),
            pl.BlockSpec((_DIM, _NEMB), lambda i: (0, 0)),
        ],
        out_specs=[
            pl.BlockSpec((_ROWS_PER_BLOCK,), lambda i: (i,)),
            pl.BlockSpec((8, 128), lambda i: (0, 0)),
        ],
        out_shape=[
            jax.ShapeDtypeStruct((n,), jnp.int32),
            jax.ShapeDtypeStruct((8, 128), jnp.float32),
        ],
        compiler_params=pltpu.CompilerParams(
            dimension_semantics=("arbitrary",)),
    )(flat_x, embeddings)


def _sc_gather(table, indices):
    n = indices.shape[0]
    width = table.shape[1]
    mesh = plsc.VectorSubcoreMesh(
        core_axis_name="core", subcore_axis_name="subcore")
    num_workers = mesh.num_cores * mesh.num_subcores
    per_worker = n // num_workers

    @pl.kernel(
        out_type=jax.ShapeDtypeStruct((n, width), jnp.float32),
        mesh=mesh,
        scratch_types=[
            pltpu.VMEM((per_worker,), jnp.int32),
            pltpu.VMEM((per_worker, width), jnp.float32),
            pltpu.SemaphoreType.DMA,
        ],
        compiler_params=pltpu.CompilerParams(use_tc_tiling_on_sc=False),
    )
    def gather_kernel(tab_hbm, i_hbm, o_hbm, idx_v, rows_v, sem):
        wid = (jax.lax.axis_index("subcore") * mesh.num_cores
               + jax.lax.axis_index("core"))
        base = wid * per_worker
        pltpu.sync_copy(i_hbm.at[pl.ds(base, per_worker)], idx_v)
        pltpu.async_copy(tab_hbm.at[idx_v], rows_v, sem).wait()
        pltpu.sync_copy(rows_v, o_hbm.at[pl.ds(base, per_worker)])

    return gather_kernel(table, indices)


def kernel(x, embeddings):
    n = x.shape[0] * x.shape[1]
    flat_x = x.reshape(n, _DIM)
    idx, acc = _distance_argmin(flat_x, embeddings)
    loss = acc[0, 0] * (1.25 / (n * _DIM))
    quantized = _sc_gather(embeddings.T, idx)
    return quantized.reshape(x.shape), loss


# speedup vs baseline: 1.3409x; 1.0455x over previous
import jax
import jax.numpy as jnp
from jax.experimental import pallas as pl
from jax.experimental.pallas import tpu as pltpu
from jax.experimental.pallas import tpu_sc as plsc

_DIM = 64
_NEMB = 1024
_ROWS_PER_BLOCK = 512
_GATHER_WINDOW = 256


_BIG = 3.0e38


def _distance_argmin_body(x_ref, e_ref, idx_ref, acc_ref):
    i = pl.program_id(0)
    xb = x_ref[...]
    emb = e_ref[...]
    xm2 = xb * -2.0
    x2 = jnp.sum(xb * xb, axis=1, keepdims=True)
    e2 = jnp.sum(emb * emb, axis=0, keepdims=True)
    x2b = jnp.broadcast_to(x2, (_ROWS_PER_BLOCK, 128))
    lane = jax.lax.broadcasted_iota(
        jnp.int32, (_ROWS_PER_BLOCK, 128), 1).astype(jnp.float32)
    run_min = jnp.full((_ROWS_PER_BLOCK, 128), _BIG, jnp.float32)
    run_idx = jnp.full((_ROWS_PER_BLOCK, 128), _BIG, jnp.float32)
    for g in range(_NEMB // 128):
        lo, hi = g * 128, (g + 1) * 128
        neg2m_g = jax.lax.dot_general(
            xm2, emb[:, lo:hi],
            dimension_numbers=(((1,), (0,)), ((), ())),
            preferred_element_type=jnp.float32,
        )
        dg = (x2b + e2[:, lo:hi]) + neg2m_g
        better = dg < run_min
        run_idx = jnp.where(better, lane + jnp.float32(g * 128), run_idx)
        run_min = jnp.minimum(dg, run_min)
    m = jnp.min(run_min, axis=1, keepdims=True)
    idxf = jnp.min(jnp.where(run_min == m, run_idx, _BIG), axis=1)
    idx_ref[...] = idxf.astype(jnp.int32)

    @pl.when(i == 0)
    def _():
        acc_ref[...] = jnp.zeros_like(acc_ref)

    acc_ref[...] += jnp.full(acc_ref.shape, jnp.sum(m), dtype=jnp.float32)


def _distance_argmin(flat_x, embeddings):
    n = flat_x.shape[0]
    nblk = n // _ROWS_PER_BLOCK
    return pl.pallas_call(
        _distance_argmin_body,
        grid=(nblk,),
        in_specs=[
            pl.BlockSpec((_ROWS_PER_BLOCK, _DIM), lambda i: (i, 0)),
            pl.BlockSpec((_DIM, _NEMB), lambda i: (0, 0)),
        ],
        out_specs=[
            pl.BlockSpec((_ROWS_PER_BLOCK,), lambda i: (i,)),
            pl.BlockSpec((8, 128), lambda i: (0, 0)),
        ],
        out_shape=[
            jax.ShapeDtypeStruct((n,), jnp.int32),
            jax.ShapeDtypeStruct((8, 128), jnp.float32),
        ],
        compiler_params=pltpu.CompilerParams(
            dimension_semantics=("arbitrary",)),
    )(flat_x, embeddings)


def _sc_gather(table, indices):
    n = indices.shape[0]
    width = table.shape[1]
    mesh = plsc.VectorSubcoreMesh(
        core_axis_name="core", subcore_axis_name="subcore")
    num_workers = mesh.num_cores * mesh.num_subcores
    per_worker = n // num_workers

    @pl.kernel(
        out_type=jax.ShapeDtypeStruct((n, width), jnp.float32),
        mesh=mesh,
        scratch_types=[
            pltpu.VMEM((per_worker,), jnp.int32),
            pltpu.VMEM((per_worker, width), jnp.float32),
            pltpu.SemaphoreType.DMA,
        ],
        compiler_params=pltpu.CompilerParams(use_tc_tiling_on_sc=False),
    )
    def gather_kernel(tab_hbm, i_hbm, o_hbm, idx_v, rows_v, sem):
        wid = (jax.lax.axis_index("subcore") * mesh.num_cores
               + jax.lax.axis_index("core"))
        base = wid * per_worker
        pltpu.sync_copy(i_hbm.at[pl.ds(base, per_worker)], idx_v)
        pltpu.async_copy(tab_hbm.at[idx_v], rows_v, sem).wait()
        pltpu.sync_copy(rows_v, o_hbm.at[pl.ds(base, per_worker)])

    return gather_kernel(table, indices)


def kernel(x, embeddings):
    n = x.shape[0] * x.shape[1]
    flat_x = x.reshape(n, _DIM)
    idx, acc = _distance_argmin(flat_x, embeddings)
    loss = acc[0, 0] * (1.25 / (n * _DIM))
    quantized = _sc_gather(embeddings.T, idx)
    return quantized.reshape(x.shape), loss


# speedup vs baseline: 1.3521x; 1.0083x over previous
import jax
import jax.numpy as jnp
from jax.experimental import pallas as pl
from jax.experimental.pallas import tpu as pltpu
from jax.experimental.pallas import tpu_sc as plsc

_DIM = 64
_NEMB = 1024
_ROWS_PER_BLOCK = 1024
_GATHER_WINDOW = 256


_BIG = 3.0e38


def _distance_argmin_body(x_ref, e_ref, idx_ref, acc_ref):
    i = pl.program_id(0)
    xb = x_ref[...]
    emb = e_ref[...]
    xm2 = xb * -2.0
    x2 = jnp.sum(xb * xb, axis=1, keepdims=True)
    e2 = jnp.sum(emb * emb, axis=0, keepdims=True)
    x2b = jnp.broadcast_to(x2, (_ROWS_PER_BLOCK, 128))
    lane = jax.lax.broadcasted_iota(
        jnp.int32, (_ROWS_PER_BLOCK, 128), 1).astype(jnp.float32)
    run_min = jnp.full((_ROWS_PER_BLOCK, 128), _BIG, jnp.float32)
    run_idx = jnp.full((_ROWS_PER_BLOCK, 128), _BIG, jnp.float32)
    for g in range(_NEMB // 128):
        lo, hi = g * 128, (g + 1) * 128
        neg2m_g = jax.lax.dot_general(
            xm2, emb[:, lo:hi],
            dimension_numbers=(((1,), (0,)), ((), ())),
            preferred_element_type=jnp.float32,
        )
        dg = (x2b + e2[:, lo:hi]) + neg2m_g
        better = dg < run_min
        run_idx = jnp.where(better, lane + jnp.float32(g * 128), run_idx)
        run_min = jnp.minimum(dg, run_min)
    m = jnp.min(run_min, axis=1, keepdims=True)
    idxf = jnp.min(jnp.where(run_min == m, run_idx, _BIG), axis=1)
    idx_ref[...] = idxf.astype(jnp.int32)

    @pl.when(i == 0)
    def _():
        acc_ref[...] = jnp.zeros_like(acc_ref)

    acc_ref[...] += jnp.full(acc_ref.shape, jnp.sum(m), dtype=jnp.float32)


def _distance_argmin(flat_x, embeddings):
    n = flat_x.shape[0]
    nblk = n // _ROWS_PER_BLOCK
    return pl.pallas_call(
        _distance_argmin_body,
        grid=(nblk,),
        in_specs=[
            pl.BlockSpec((_ROWS_PER_BLOCK, _DIM), lambda i: (i, 0)),
            pl.BlockSpec((_DIM, _NEMB), lambda i: (0, 0)),
        ],
        out_specs=[
            pl.BlockSpec((_ROWS_PER_BLOCK,), lambda i: (i,)),
            pl.BlockSpec((8, 128), lambda i: (0, 0)),
        ],
        out_shape=[
            jax.ShapeDtypeStruct((n,), jnp.int32),
            jax.ShapeDtypeStruct((8, 128), jnp.float32),
        ],
        compiler_params=pltpu.CompilerParams(
            dimension_semantics=("arbitrary",)),
    )(flat_x, embeddings)


def _sc_gather(table, indices):
    n = indices.shape[0]
    width = table.shape[1]
    mesh = plsc.VectorSubcoreMesh(
        core_axis_name="core", subcore_axis_name="subcore")
    num_workers = mesh.num_cores * mesh.num_subcores
    per_worker = n // num_workers

    @pl.kernel(
        out_type=jax.ShapeDtypeStruct((n, width), jnp.float32),
        mesh=mesh,
        scratch_types=[
            pltpu.VMEM((per_worker,), jnp.int32),
            pltpu.VMEM((per_worker, width), jnp.float32),
            pltpu.SemaphoreType.DMA,
        ],
        compiler_params=pltpu.CompilerParams(use_tc_tiling_on_sc=False),
    )
    def gather_kernel(tab_hbm, i_hbm, o_hbm, idx_v, rows_v, sem):
        wid = (jax.lax.axis_index("subcore") * mesh.num_cores
               + jax.lax.axis_index("core"))
        base = wid * per_worker
        pltpu.sync_copy(i_hbm.at[pl.ds(base, per_worker)], idx_v)
        pltpu.async_copy(tab_hbm.at[idx_v], rows_v, sem).wait()
        pltpu.sync_copy(rows_v, o_hbm.at[pl.ds(base, per_worker)])

    return gather_kernel(table, indices)


def kernel(x, embeddings):
    n = x.shape[0] * x.shape[1]
    flat_x = x.reshape(n, _DIM)
    idx, acc = _distance_argmin(flat_x, embeddings)
    loss = acc[0, 0] * (1.25 / (n * _DIM))
    quantized = _sc_gather(embeddings.T, idx)
    return quantized.reshape(x.shape), loss
